# baseline (speedup 1.0000x reference)
import jax
import jax.numpy as jnp
from jax.experimental import pallas as pl

_EPS = 1e-5


def _body(a_ref, w_ref, bias_ref, g_ref, beta_ref, o_ref):
    w = w_ref[...]
    bb = bias_ref[...]
    g = g_ref[...]
    beta = beta_ref[...]
    wbar = jnp.mean(w)
    bbar = jnp.mean(bb)
    dw = w - wbar
    db = bb - bbar
    p = jnp.mean(dw * dw)
    q = jnp.mean(dw * db)
    r = jnp.mean(db * db)
    va = dw * g
    vb = db * g
    a = a_ref[0, 0]
    s = jax.lax.rsqrt((a * a) * p + a * (2.0 * q) + (r + _EPS))
    o_ref[0, 0] = (a * s) * va + s * vb + beta


def kernel(data, time, weekday, W, b, ln_gamma, ln_beta):
    del time, weekday
    bsz, seq, nodes, _ = data.shape
    size = W.shape[0]
    row = lambda v: v.reshape(1, size)
    vec_spec = pl.BlockSpec((1, size), lambda i, j: (0, 0))
    return pl.pallas_call(
        _body,
        grid=(bsz, seq),
        in_specs=[
            pl.BlockSpec((1, 1, nodes, 1), lambda i, j: (i, j, 0, 0)),
            vec_spec, vec_spec, vec_spec, vec_spec,
        ],
        out_specs=pl.BlockSpec((1, 1, nodes, size), lambda i, j: (i, j, 0, 0)),
        out_shape=jax.ShapeDtypeStruct((bsz, seq, nodes, size), jnp.float32),
    )(data, row(W), row(b), row(ln_gamma), row(ln_beta))


# speedup vs baseline: 2.3425x; 2.3425x over previous
import jax
import jax.numpy as jnp
from jax.experimental import pallas as pl

_EPS = 1e-5


def _body(a_ref, w_ref, bias_ref, g_ref, beta_ref, o_ref):
    w = w_ref[...]
    bb = bias_ref[...]
    g = g_ref[...]
    beta = beta_ref[...]
    wbar = jnp.mean(w)
    bbar = jnp.mean(bb)
    dw = w - wbar
    db = bb - bbar
    p = jnp.mean(dw * dw)
    q = jnp.mean(dw * db)
    r = jnp.mean(db * db)
    va = dw * g
    vb = db * g
    arow = a_ref[0]
    s = jax.lax.rsqrt((arow * arow) * p + arow * (2.0 * q) + (r + _EPS))
    c1 = (arow * s)[:, :, None]
    c2 = s[:, :, None]
    o_ref[0] = c1 * va[None] + c2 * vb[None] + beta[None]


def kernel(data, time, weekday, W, b, ln_gamma, ln_beta):
    del time, weekday
    bsz, seq, nodes, _ = data.shape
    size = W.shape[0]
    row = lambda v: v.reshape(1, size)
    vec_spec = pl.BlockSpec((1, size), lambda i: (0, 0))
    return pl.pallas_call(
        _body,
        grid=(bsz,),
        in_specs=[
            pl.BlockSpec((1, seq, nodes), lambda i: (i, 0, 0)),
            vec_spec, vec_spec, vec_spec, vec_spec,
        ],
        out_specs=pl.BlockSpec((1, seq, nodes, size), lambda i: (i, 0, 0, 0)),
        out_shape=jax.ShapeDtypeStruct((bsz, seq, nodes, size), jnp.float32),
    )(data.reshape(bsz, seq, nodes), row(W), row(b), row(ln_gamma), row(ln_beta))
